# baseline (speedup 1.0000x reference)
import functools

import jax
import jax.numpy as jnp
from jax import lax
from jax.experimental import pallas as pl
from jax.experimental.pallas import tpu as pltpu
from jax.experimental.pallas import tpu_sc as plsc

N_NODES = 10000
N_EDGES = 160000
D = 256
H = D // 2
NS = 16
NL = 16
EPT = N_EDGES // NS
B = 128
KR = 79
KB = 88
NPAD = 10112
RPT = NPAD // NS


def _sc_segment_sum(x2, idx_packed, zer):
    mesh = plsc.VectorSubcoreMesh(core_axis_name="c", subcore_axis_name="s")

    @functools.partial(
        pl.kernel,
        out_type=jax.ShapeDtypeStruct((2, NPAD, H), jnp.float32),
        mesh=mesh,
        scratch_types=[
            pltpu.VMEM((B,), jnp.int32),
            pltpu.VMEM((B,), jnp.int32),
            pltpu.VMEM((B,), jnp.int32),
            pltpu.VMEM((B,), jnp.int32),
            pltpu.VMEM((B,), jnp.int32),
            pltpu.VMEM((B,), jnp.int32),
            pltpu.VMEM((B,), jnp.int32),
            pltpu.VMEM((B, H), jnp.float32),
            pltpu.VMEM((B, H), jnp.float32),
            pltpu.VMEM((B, H), jnp.float32),
            pltpu.VMEM_SHARED((NPAD, H), jnp.float32),
            pltpu.SemaphoreType.DMA,
            pltpu.SemaphoreType.DMA,
            pltpu.SemaphoreType.DMA,
            pltpu.SemaphoreType.DMA,
            pltpu.SemaphoreType.DMA,
            pltpu.SemaphoreType.DMA,
            pltpu.SemaphoreType.DMA,
            pltpu.SemaphoreType.DMA,
        ],
    )
    def sc_accum(x2_hbm, idx_hbm, zer_hbm, out_hbm,
                 ib0, ib1, ga0, ga1, ra0, ra1, ra2, buf0, buf1, buf2, s_sh,
                 si0, si1, sg0, sg1, sg2, ss0, ss1, ss2):
        c = lax.axis_index("c")
        t = lax.axis_index("s")
        ib = (ib0, ib1)
        si = (si0, si1)
        ga = (ga0, ga1)
        ra = (ra0, ra1, ra2)
        buf = (buf0, buf1, buf2)
        sg = (sg0, sg1, sg2)
        ss = (ss0, ss1, ss2)
        pltpu.sync_copy(zer_hbm, s_sh.at[pl.ds(t * RPT, RPT)])
        plsc.subcore_barrier()

        def unpack(b, g, r):
            for v in range(B // NL):
                w = b[pl.ds(NL * v, NL)]
                g[pl.ds(NL * v, NL)] = (w & 0x1FFFF) + c
                r[pl.ds(NL * v, NL)] = lax.shift_right_logical(w, 17)

        def ring_step(k, i2, i3, g2, g3, d1, d2, d3, with_prev_wait=True):
            pltpu.make_async_copy(idx_hbm.at[t, k + 1], ib[i2], si[i2]).wait()
            unpack(ib[i2], ga[g2], ra[d2])
            pltpu.async_copy(x2_hbm.at[ga[g2]], buf[d2], sg[d2])
            pltpu.make_async_copy(x2_hbm.at[ga[g3]], buf[d1], sg[d1]).wait()
            pltpu.async_copy(buf[d1], s_sh.at[ra[d1]], ss[d1], add=True)
            if with_prev_wait:
                pltpu.make_async_copy(buf[d3], s_sh.at[ra[d3]], ss[d3]).wait()
            pltpu.async_copy(idx_hbm.at[t, k + 2], ib[i3], si[i3])

        pltpu.sync_copy(idx_hbm.at[t, 0], ib0)
        unpack(ib0, ga0, ra0)
        pltpu.async_copy(x2_hbm.at[ga0], buf0, sg0)
        pltpu.async_copy(idx_hbm.at[t, 1], ib1, si1)
        ring_step(0, 1, 0, 1, 0, 0, 1, 2, with_prev_wait=False)

        def step(j, carry):
            k0 = 6 * j + 1
            ring_step(k0, 0, 1, 0, 1, 1, 2, 0)
            ring_step(k0 + 1, 1, 0, 1, 0, 2, 0, 1)
            ring_step(k0 + 2, 0, 1, 0, 1, 0, 1, 2)
            ring_step(k0 + 3, 1, 0, 1, 0, 1, 2, 0)
            ring_step(k0 + 4, 0, 1, 0, 1, 2, 0, 1)
            ring_step(k0 + 5, 1, 0, 1, 0, 0, 1, 2)
            return carry

        lax.fori_loop(0, (KR - 1) // 6, step, 0)
        pltpu.make_async_copy(x2_hbm.at[ga[KR % 2]], buf[KR % 3],
                              sg[KR % 3]).wait()
        pltpu.make_async_copy(buf[(KR - 1) % 3], s_sh.at[ra[(KR - 1) % 3]],
                              ss[(KR - 1) % 3]).wait()
        pltpu.make_async_copy(idx_hbm.at[t, KR + 1], ib[(KR + 1) % 2],
                              si[(KR + 1) % 2]).wait()
        plsc.subcore_barrier()
        pltpu.sync_copy(s_sh.at[pl.ds(t * RPT, RPT)],
                        out_hbm.at[c, pl.ds(t * RPT, RPT)])

    return sc_accum(x2, idx_packed, zer)


def _tc_finish(x, s0, s1, wt, bias2):
    blk = 2000
    grid = (N_NODES // blk,)

    def body(x_ref, s0_ref, s1_ref, wt_ref, b_ref, o_ref):
        xs = x_ref[...] * jnp.concatenate([s0_ref[...], s1_ref[...]], axis=-1)
        o_ref[...] = (jnp.dot(xs, wt_ref[...],
                              preferred_element_type=jnp.float32)
                      + b_ref[...])

    return pl.pallas_call(
        body,
        grid=grid,
        in_specs=[
            pl.BlockSpec((blk, D), lambda i: (i, 0)),
            pl.BlockSpec((blk, H), lambda i: (i, 0)),
            pl.BlockSpec((blk, H), lambda i: (i, 0)),
            pl.BlockSpec((D, D), lambda i: (0, 0)),
            pl.BlockSpec((1, D), lambda i: (0, 0)),
        ],
        out_specs=pl.BlockSpec((blk, D), lambda i: (i, 0)),
        out_shape=jax.ShapeDtypeStruct((N_NODES, D), jnp.float32),
    )(x, s0, s1, wt, bias2)


def kernel(x, edge_index, weight, bias):
    row = edge_index[0].astype(jnp.int32)
    col = edge_index[1].astype(jnp.int32)
    x2 = x.reshape(2 * N_NODES, H)
    packed = (row << 17) | (col << 1)
    pad = jnp.full((NS, KB * B - EPT), (NPAD - 1) << 17, jnp.int32)
    idx_packed = jnp.concatenate(
        [packed.reshape(NS, EPT), pad], axis=1).reshape(NS, KB, B)
    zer = jnp.zeros((RPT, H), dtype=jnp.float32)

    s = _sc_segment_sum(x2, idx_packed, zer)

    wt = weight.T
    bias2 = bias[None, :]
    return _tc_finish(x, s[0], s[1], wt, bias2)


# speedup vs baseline: 1.3398x; 1.3398x over previous
import functools

import jax
import jax.numpy as jnp
from jax import lax
from jax.experimental import pallas as pl
from jax.experimental.pallas import tpu as pltpu
from jax.experimental.pallas import tpu_sc as plsc

N_NODES = 10000
N_EDGES = 160000
D = 256
H = D // 2
NS = 16
NL = 16
EPT = N_EDGES // NS
B = 128
KR = 79
KB = KR + 1
NPAD = 10112
RPT = NPAD // NS


def _sc_segment_sum(x2, idx_packed, zer):
    mesh = plsc.VectorSubcoreMesh(core_axis_name="c", subcore_axis_name="s")

    @functools.partial(
        pl.kernel,
        out_type=jax.ShapeDtypeStruct((2, NPAD, H), jnp.float32),
        mesh=mesh,
        scratch_types=[
            pltpu.VMEM((B,), jnp.int32),
            pltpu.VMEM((B,), jnp.int32),
            pltpu.VMEM((B,), jnp.int32),
            pltpu.VMEM((B,), jnp.int32),
            pltpu.VMEM((B,), jnp.int32),
            pltpu.VMEM((B,), jnp.int32),
            pltpu.VMEM((B, H), jnp.float32),
            pltpu.VMEM((B, H), jnp.float32),
            pltpu.VMEM_SHARED((NPAD, H), jnp.float32),
            pltpu.SemaphoreType.DMA,
            pltpu.SemaphoreType.DMA,
            pltpu.SemaphoreType.DMA,
            pltpu.SemaphoreType.DMA,
        ],
    )
    def sc_accum(x2_hbm, idx_hbm, zer_hbm, out_hbm,
                 ib0, ib1, ga0, ga1, ra0, ra1, buf0, buf1, s_sh,
                 si0, si1, sg0, sg1):
        c = lax.axis_index("c")
        t = lax.axis_index("s")
        pltpu.sync_copy(zer_hbm, s_sh.at[pl.ds(t * RPT, RPT)])
        plsc.subcore_barrier()

        def unpack(ib, ga, ra):
            for v in range(B // NL):
                w = ib[pl.ds(NL * v, NL)]
                ga[pl.ds(NL * v, NL)] = (w & 0x1FFFF) + c
                ra[pl.ds(NL * v, NL)] = lax.shift_right_logical(w, 17)

        pltpu.sync_copy(idx_hbm.at[t, 0], ib0)
        unpack(ib0, ga0, ra0)
        pltpu.async_copy(x2_hbm.at[ga0], buf0, sg0)
        pltpu.async_copy(idx_hbm.at[t, 1], ib1, si1)

        def half_step(k, ib_a, si_a, ga_a, ra_a, buf_a, sg_a,
                      ib_b, si_b, ga_b, ra_b, buf_b, sg_b):
            pltpu.make_async_copy(idx_hbm.at[t, k + 1], ib_b, si_b).wait()
            unpack(ib_b, ga_b, ra_b)
            pltpu.async_copy(x2_hbm.at[ga_b], buf_b, sg_b)
            pltpu.async_copy(idx_hbm.at[t, k + 2], ib_a, si_a)
            pltpu.make_async_copy(x2_hbm.at[ga_a], buf_a, sg_a).wait()
            pltpu.sync_copy(buf_a, s_sh.at[ra_a], add=True)

        def step(j, carry):
            k0 = 2 * j
            half_step(k0, ib0, si0, ga0, ra0, buf0, sg0,
                      ib1, si1, ga1, ra1, buf1, sg1)
            half_step(k0 + 1, ib1, si1, ga1, ra1, buf1, sg1,
                      ib0, si0, ga0, ra0, buf0, sg0)
            return carry

        lax.fori_loop(0, (KR - 1) // 2, step, 0)
        pltpu.make_async_copy(x2_hbm.at[ga0], buf0, sg0).wait()
        pltpu.sync_copy(buf0, s_sh.at[ra0], add=True)
        pltpu.make_async_copy(idx_hbm.at[t, KR], ib1, si1).wait()
        plsc.subcore_barrier()
        pltpu.sync_copy(s_sh.at[pl.ds(t * RPT, RPT)],
                        out_hbm.at[c, pl.ds(t * RPT, RPT)])

    return sc_accum(x2, idx_packed, zer)


def _tc_finish(x, s0, s1, wt, bias2):
    blk = 2000
    grid = (N_NODES // blk,)

    def body(x_ref, s0_ref, s1_ref, wt_ref, b_ref, o_ref):
        xs = x_ref[...] * jnp.concatenate([s0_ref[...], s1_ref[...]], axis=-1)
        o_ref[...] = (jnp.dot(xs, wt_ref[...],
                              preferred_element_type=jnp.float32)
                      + b_ref[...])

    return pl.pallas_call(
        body,
        grid=grid,
        in_specs=[
            pl.BlockSpec((blk, D), lambda i: (i, 0)),
            pl.BlockSpec((blk, H), lambda i: (i, 0)),
            pl.BlockSpec((blk, H), lambda i: (i, 0)),
            pl.BlockSpec((D, D), lambda i: (0, 0)),
            pl.BlockSpec((1, D), lambda i: (0, 0)),
        ],
        out_specs=pl.BlockSpec((blk, D), lambda i: (i, 0)),
        out_shape=jax.ShapeDtypeStruct((N_NODES, D), jnp.float32),
    )(x, s0, s1, wt, bias2)


def kernel(x, edge_index, weight, bias):
    row = edge_index[0].astype(jnp.int32)
    col = edge_index[1].astype(jnp.int32)
    x2 = x.reshape(2 * N_NODES, H)
    packed = (row << 17) | (col << 1)
    pad = jnp.full((NS, KB * B - EPT), (NPAD - 1) << 17, jnp.int32)
    idx_packed = jnp.concatenate(
        [packed.reshape(NS, EPT), pad], axis=1).reshape(NS, KB, B)
    zer = jnp.zeros((RPT, H), dtype=jnp.float32)

    s = _sc_segment_sum(x2, idx_packed, zer)

    wt = weight.T
    bias2 = bias[None, :]
    return _tc_finish(x, s[0], s[1], wt, bias2)
